# Initial kernel scaffold; baseline (speedup 1.0000x reference)
import functools

import jax
import jax.numpy as jnp
from jax import lax
from jax.experimental import pallas as pl
from jax.experimental.pallas import tpu as pltpu
from jax.experimental.pallas import tpu_sc as plsc

_B, _H, _W, _K, _P, _C = 2, 224, 224, 8, 100000, 128
_N = _B * _H * _W
_NW = 32
_NPW = _N // _NW
_CHUNK = 16
_NCHUNK = _NPW // _CHUNK
_ROWS_PER_WORKER = _NPW * _K // 128

_IOTA16 = jnp.arange(16, dtype=jnp.int32)


def _take16(v, idx):
    return jnp.take(v, idx, mode="promise_in_bounds")


def _sc_body(feat_hbm, idx_hbm, dists_hbm, depth_hbm, out_hbm,
             idx_v, dist_v, depth_v, rows_v, out_v, sem):
    nc = 2
    wid = lax.axis_index("s") * nc + lax.axis_index("c")
    row0 = wid * _ROWS_PER_WORKER
    pix0 = wid * _NPW

    pltpu.sync_copy(idx_hbm.at[pl.ds(row0, _ROWS_PER_WORKER)], idx_v)
    pltpu.sync_copy(dists_hbm.at[pl.ds(row0, _ROWS_PER_WORKER)], dist_v)
    pltpu.sync_copy(depth_hbm.at[pl.ds(pix0, _NPW)], depth_v)

    def chunk_body(g, carry):
        pltpu.async_copy(feat_hbm.at[idx_v.at[g]], rows_v, sem).wait()

        winvs = []
        for j in range(8):
            d = dist_v[g, pl.ds(j * 16, 16)]
            w = 1.0 - d
            s = w
            for sh in (1, 2, 4):
                s = s + _take16(s, _IOTA16 ^ sh)
            winvs.append(w / jnp.maximum(s, 1e-10))

        for p in range(16):
            wv = winvs[p // 2]
            base = (p % 2) * 8
            accs = None
            for k in range(8):
                ws = _take16(wv, jnp.full((16,), base + k, jnp.int32))
                r = p * 8 + k
                terms = [ws * rows_v[r, pl.ds(cb * 16, 16)] for cb in range(8)]
                accs = terms if accs is None else [a + t for a, t in zip(accs, terms)]
            for cb in range(8):
                out_v[p, pl.ds(cb * 16, 16)] = accs[cb]

        dvec = depth_v[pl.ds(g * _CHUNK, _CHUNK)]
        plsc.store_scatter(out_v, [_IOTA16, jnp.full((16,), _C, jnp.int32)], dvec)

        pltpu.sync_copy(out_v, out_hbm.at[pl.ds(pix0 + g * _CHUNK, _CHUNK)])
        return carry

    lax.fori_loop(0, _NCHUNK, chunk_body, 0)


_sc_render = functools.partial(
    pl.kernel,
    mesh=plsc.VectorSubcoreMesh(core_axis_name="c", subcore_axis_name="s"),
    out_type=jax.ShapeDtypeStruct((_N, _C + 1), jnp.float32),
    scratch_types=[
        pltpu.VMEM((_ROWS_PER_WORKER, 128), jnp.int32),
        pltpu.VMEM((_ROWS_PER_WORKER, 128), jnp.float32),
        pltpu.VMEM((_NPW,), jnp.float32),
        pltpu.VMEM((_CHUNK * _K, _C), jnp.float32),
        pltpu.VMEM((_CHUNK, _C + 1), jnp.float32),
        pltpu.SemaphoreType.DMA,
    ],
)(_sc_body)


def _depth_body(z_ref, o_ref):
    x = z_ref[...]
    m = x == -1.0
    ne = jnp.sum(m.astype(jnp.float32), axis=1, keepdims=True)
    dsum = jnp.sum(x, axis=1, keepdims=True) + ne
    mean = dsum / (float(_H * _W) - ne)
    o_ref[...] = jnp.where(m, -1.0, x - mean)


def _depth_normalize(depth_raw):
    return pl.pallas_call(
        _depth_body,
        out_shape=jax.ShapeDtypeStruct((_B, _H * _W), jnp.float32),
    )(depth_raw)


def kernel(idx, zbuf, dists, features):
    idx2 = idx.astype(jnp.int32).reshape(_N * _K // 128, 128)
    d2 = dists.reshape(_N * _K // 128, 128)
    depth_raw = zbuf[..., 0].reshape(_B, _H * _W)
    depth_n = _depth_normalize(depth_raw).reshape(_N)
    out = _sc_render(features, idx2, d2, depth_n)
    return out.reshape(_B, _H, _W, _C + 1)


# speedup vs baseline: 2.5561x; 2.5561x over previous
import functools

import jax
import jax.numpy as jnp
import numpy as np
from jax import lax
from jax.experimental import pallas as pl
from jax.experimental.pallas import tpu as pltpu
from jax.experimental.pallas import tpu_sc as plsc

_B, _H, _W, _K, _P, _C = 2, 224, 224, 8, 100000, 128
_N = _B * _H * _W
_NW = 32
_NPW = _N // _NW
_CHUNK = 16
_NCHUNK = _NPW // _CHUNK
_ROWS_PER_WORKER = _NPW * _K // 128

def _take16(v, idx):
    return v.at[idx].get(mode="promise_in_bounds")


def _sc_body(feat_hbm, idx_hbm, dists_hbm, depth_hbm, out_hbm,
             idx_v, dist_v, depth_v, rows_v, out_v, sem):
    nc = 2
    iota16 = lax.broadcasted_iota(jnp.int32, (16,), 0)
    wid = lax.axis_index("s") * nc + lax.axis_index("c")
    elem0 = pl.multiple_of(wid * (_NPW * _K), 8)
    pix0 = pl.multiple_of(wid * _NPW, 8)

    pltpu.sync_copy(idx_hbm.at[pl.ds(elem0, _NPW * _K)], idx_v)
    pltpu.sync_copy(dists_hbm.at[pl.ds(elem0, _NPW * _K)], dist_v)
    pltpu.sync_copy(depth_hbm.at[pl.ds(pix0, _NPW)], depth_v)

    def chunk_body(g, carry):
        goff = pl.multiple_of(g * (_CHUNK * _K), 8)
        pltpu.async_copy(
            feat_hbm.at[idx_v.at[pl.ds(goff, _CHUNK * _K)]], rows_v, sem
        ).wait()

        winvs = []
        for j in range(8):
            d = dist_v[pl.ds(goff + j * 16, 16)]
            w = 1.0 - d
            s = w
            for sh in (1, 2, 4):
                s = s + _take16(s, iota16 ^ sh)
            winvs.append(w / jnp.maximum(s, 1e-10))

        dvec = depth_v[pl.ds(g * _CHUNK, _CHUNK)]
        for p in range(16):
            dsplat = _take16(dvec, jnp.zeros_like(iota16) + p)
            out_v[pl.ds(p * (_C + 1) + _C, 16)] = dsplat

        for p in range(16):
            wv = winvs[p // 2]
            base = (p % 2) * 8
            accs = None
            for k in range(8):
                ws = _take16(wv, jnp.zeros_like(iota16) + (base + k))
                r = p * 8 + k
                terms = [ws * rows_v[r, pl.ds(cb * 16, 16)] for cb in range(8)]
                accs = terms if accs is None else [a + t for a, t in zip(accs, terms)]
            for cb in range(8):
                out_v[pl.ds(p * (_C + 1) + cb * 16, 16)] = accs[cb]

        obase = pl.multiple_of((pix0 + g * _CHUNK) * (_C + 1), 8)
        pltpu.sync_copy(
            out_v.at[pl.ds(0, _CHUNK * (_C + 1))],
            out_hbm.at[pl.ds(obase, _CHUNK * (_C + 1))],
        )
        return carry

    lax.fori_loop(0, _NCHUNK, chunk_body, 0)


_sc_render = functools.partial(
    pl.kernel,
    mesh=plsc.VectorSubcoreMesh(core_axis_name="c", subcore_axis_name="s"),
    out_type=jax.ShapeDtypeStruct((_N * (_C + 1),), jnp.float32),
    scratch_types=[
        pltpu.VMEM((_NPW * _K,), jnp.int32),
        pltpu.VMEM((_NPW * _K,), jnp.float32),
        pltpu.VMEM((_NPW,), jnp.float32),
        pltpu.VMEM((_CHUNK * _K, _C), jnp.float32),
        pltpu.VMEM((_CHUNK * (_C + 1) + 16,), jnp.float32),
        pltpu.SemaphoreType.DMA,
    ],
)(_sc_body)


def _depth_body(z_ref, o_ref):
    x = z_ref[...]
    m = x == -1.0
    ne = jnp.sum(m.astype(jnp.float32), axis=1, keepdims=True)
    dsum = jnp.sum(x, axis=1, keepdims=True) + ne
    mean = dsum / (float(_H * _W) - ne)
    o_ref[...] = jnp.where(m, -1.0, x - mean)


def _depth_normalize(depth_raw):
    return pl.pallas_call(
        _depth_body,
        out_shape=jax.ShapeDtypeStruct((_B, _H * _W), jnp.float32),
    )(depth_raw)


def kernel(idx, zbuf, dists, features):
    idx_flat = idx.astype(jnp.int32).reshape(_N * _K)
    d_flat = dists.reshape(_N * _K)
    depth_raw = zbuf[..., 0].reshape(_B, _H * _W)
    depth_n = _depth_normalize(depth_raw).reshape(_N)
    out = _sc_render(features, idx_flat, d_flat, depth_n)
    return out.reshape(_B, _H, _W, _C + 1)


# speedup vs baseline: 3.0217x; 1.1822x over previous
import functools

import jax
import jax.numpy as jnp
import numpy as np
from jax import lax
from jax.experimental import pallas as pl
from jax.experimental.pallas import tpu as pltpu
from jax.experimental.pallas import tpu_sc as plsc

_B, _H, _W, _K, _P, _C = 2, 224, 224, 8, 100000, 128
_N = _B * _H * _W
_NW = 32
_NPW = _N // _NW
_CHUNK = 16
_NCHUNK = _NPW // _CHUNK
_ROWS_PER_WORKER = _NPW * _K // 128

def _take16(v, idx):
    return v.at[idx].get(mode="promise_in_bounds")


def _sc_body(feat_hbm, idx_hbm, dists_hbm, depth_hbm, out_hbm,
             idx_v, dist_v, depth_v, rows_a, rows_b, out_v, sem_a, sem_b):
    nc = 2
    iota16 = lax.broadcasted_iota(jnp.int32, (16,), 0)
    wid = lax.axis_index("s") * nc + lax.axis_index("c")
    elem0 = pl.multiple_of(wid * (_NPW * _K), 8)
    pix0 = pl.multiple_of(wid * _NPW, 8)

    pltpu.sync_copy(idx_hbm.at[pl.ds(elem0, _NPW * _K)], idx_v)
    pltpu.sync_copy(dists_hbm.at[pl.ds(elem0, _NPW * _K)], dist_v)
    pltpu.sync_copy(depth_hbm.at[pl.ds(pix0, _NPW)], depth_v)

    def fire(g, rows, sem):
        goff = pl.multiple_of(g * (_CHUNK * _K), 8)
        pltpu.make_async_copy(
            feat_hbm.at[idx_v.at[pl.ds(goff, _CHUNK * _K)]], rows, sem
        ).start()

    def compute(g, rows_v):
        goff = pl.multiple_of(g * (_CHUNK * _K), 8)
        winvs = []
        for j in range(8):
            d = dist_v[pl.ds(goff + j * 16, 16)]
            w = 1.0 - d
            s = w
            for sh in (1, 2, 4):
                s = s + _take16(s, iota16 ^ sh)
            winvs.append(w / jnp.maximum(s, 1e-10))

        dvec = depth_v[pl.ds(g * _CHUNK, _CHUNK)]
        for p in range(16):
            dsplat = _take16(dvec, jnp.zeros_like(iota16) + p)
            out_v[pl.ds(p * (_C + 1) + _C, 16)] = dsplat

        for p in range(16):
            wv = winvs[p // 2]
            base = (p % 2) * 8
            accs = None
            for k in range(8):
                ws = _take16(wv, jnp.zeros_like(iota16) + (base + k))
                r = p * 8 + k
                terms = [ws * rows_v[r, pl.ds(cb * 16, 16)] for cb in range(8)]
                accs = terms if accs is None else [a + t for a, t in zip(accs, terms)]
            for cb in range(8):
                out_v[pl.ds(p * (_C + 1) + cb * 16, 16)] = accs[cb]

        obase = pl.multiple_of((pix0 + g * _CHUNK) * (_C + 1), 8)
        pltpu.sync_copy(
            out_v.at[pl.ds(0, _CHUNK * (_C + 1))],
            out_hbm.at[pl.ds(obase, _CHUNK * (_C + 1))],
        )

    fire(0, rows_a, sem_a)
    fire(1, rows_b, sem_b)

    def pair_body(h, carry):
        for half, (rows, sem) in enumerate(((rows_a, sem_a), (rows_b, sem_b))):
            g = 2 * h + half
            pltpu.make_async_copy(
                feat_hbm.at[idx_v.at[pl.ds(0, _CHUNK * _K)]], rows, sem
            ).wait()
            compute(g, rows)

            @pl.when(h < _NCHUNK // 2 - 1)
            def _():
                fire(g + 2, rows, sem)

        return carry

    lax.fori_loop(0, _NCHUNK // 2, pair_body, 0)


_sc_render = functools.partial(
    pl.kernel,
    mesh=plsc.VectorSubcoreMesh(core_axis_name="c", subcore_axis_name="s"),
    out_type=jax.ShapeDtypeStruct((_N * (_C + 1),), jnp.float32),
    scratch_types=[
        pltpu.VMEM((_NPW * _K,), jnp.int32),
        pltpu.VMEM((_NPW * _K,), jnp.float32),
        pltpu.VMEM((_NPW,), jnp.float32),
        pltpu.VMEM((_CHUNK * _K, _C), jnp.float32),
        pltpu.VMEM((_CHUNK * _K, _C), jnp.float32),
        pltpu.VMEM((_CHUNK * (_C + 1) + 16,), jnp.float32),
        pltpu.SemaphoreType.DMA,
        pltpu.SemaphoreType.DMA,
    ],
)(_sc_body)


def _depth_body(z_ref, o_ref):
    x = z_ref[...]
    m = x == -1.0
    ne = jnp.sum(m.astype(jnp.float32), axis=1, keepdims=True)
    dsum = jnp.sum(x, axis=1, keepdims=True) + ne
    mean = dsum / (float(_H * _W) - ne)
    o_ref[...] = jnp.where(m, -1.0, x - mean)


def _depth_normalize(depth_raw):
    return pl.pallas_call(
        _depth_body,
        out_shape=jax.ShapeDtypeStruct((_B, _H * _W), jnp.float32),
    )(depth_raw)


def kernel(idx, zbuf, dists, features):
    idx_flat = idx.astype(jnp.int32).reshape(_N * _K)
    d_flat = dists.reshape(_N * _K)
    depth_raw = zbuf[..., 0].reshape(_B, _H * _W)
    depth_n = _depth_normalize(depth_raw).reshape(_N)
    out = _sc_render(features, idx_flat, d_flat, depth_n)
    return out.reshape(_B, _H, _W, _C + 1)


# speedup vs baseline: 4.7779x; 1.5812x over previous
import functools

import jax
import jax.numpy as jnp
import numpy as np
from jax import lax
from jax.experimental import pallas as pl
from jax.experimental.pallas import tpu as pltpu
from jax.experimental.pallas import tpu_sc as plsc

_B, _H, _W, _K, _P, _C = 2, 224, 224, 8, 100000, 128
_N = _B * _H * _W
_NW = 32
_NPW = _N // _NW
_CHUNK = 16
_NCHUNK = _NPW // _CHUNK
_ROWS_PER_WORKER = _NPW * _K // 128
_NBUF = 4

def _take16(v, idx):
    return v.at[idx].get(mode="promise_in_bounds")


def _sc_body(feat_hbm, idx_hbm, dists_hbm, out_hbm,
             idx_v, dist_v, rows_v, out_v, sems):
    nc = 2
    iota16 = lax.broadcasted_iota(jnp.int32, (16,), 0)
    wid = lax.axis_index("s") * nc + lax.axis_index("c")
    elem0 = pl.multiple_of(wid * (_NPW * _K), 8)
    pix0 = pl.multiple_of(wid * _NPW, 8)

    pltpu.sync_copy(idx_hbm.at[pl.ds(elem0, _NPW * _K)], idx_v)
    pltpu.sync_copy(dists_hbm.at[pl.ds(elem0, _NPW * _K)], dist_v)

    def fire(g, b):
        goff = pl.multiple_of(g * (_CHUNK * _K), 8)
        pltpu.make_async_copy(
            feat_hbm.at[idx_v.at[pl.ds(goff, _CHUNK * _K)]],
            rows_v.at[b], sems.at[b]
        ).start()

    def compute(g, b):
        goff = pl.multiple_of(g * (_CHUNK * _K), 8)
        winvs = []
        for j in range(8):
            d = dist_v[pl.ds(goff + j * 16, 16)]
            w = 1.0 - d
            s = w
            for sh in (1, 2, 4):
                s = s + _take16(s, iota16 ^ sh)
            winvs.append(w / jnp.maximum(s, 1e-10))

        for p in range(16):
            wv = winvs[p // 2]
            base = (p % 2) * 8
            accs = None
            for k in range(8):
                ws = _take16(wv, jnp.zeros_like(iota16) + (base + k))
                r = p * 8 + k
                terms = [ws * rows_v[b, r, pl.ds(cb * 16, 16)] for cb in range(8)]
                accs = terms if accs is None else [a + t for a, t in zip(accs, terms)]
            for cb in range(8):
                out_v[pl.ds(p * _C + cb * 16, 16)] = accs[cb]

        obase = pl.multiple_of((pix0 + g * _CHUNK) * _C, 8)
        pltpu.sync_copy(out_v, out_hbm.at[pl.ds(obase, _CHUNK * _C)])

    for b in range(_NBUF):
        fire(b, b)

    def ring_body(g, carry):
        b = lax.rem(g, _NBUF)
        pltpu.make_async_copy(
            feat_hbm.at[idx_v.at[pl.ds(0, _CHUNK * _K)]],
            rows_v.at[b], sems.at[b]
        ).wait()
        compute(g, b)

        @pl.when(g + _NBUF < _NCHUNK)
        def _():
            fire(g + _NBUF, b)

        return carry

    lax.fori_loop(0, _NCHUNK, ring_body, 0)


_sc_render = functools.partial(
    pl.kernel,
    mesh=plsc.VectorSubcoreMesh(core_axis_name="c", subcore_axis_name="s"),
    out_type=jax.ShapeDtypeStruct((_N * _C,), jnp.float32),
    scratch_types=[
        pltpu.VMEM((_NPW * _K,), jnp.int32),
        pltpu.VMEM((_NPW * _K,), jnp.float32),
        pltpu.VMEM((_NBUF, _CHUNK * _K, _C), jnp.float32),
        pltpu.VMEM((_CHUNK * _C,), jnp.float32),
        pltpu.SemaphoreType.DMA((_NBUF,)),
    ],
)(_sc_body)


def _depth_body(z_ref, o_ref):
    x = z_ref[...]
    m = x == -1.0
    ne = jnp.sum(m.astype(jnp.float32), axis=1, keepdims=True)
    dsum = jnp.sum(x, axis=1, keepdims=True) + ne
    mean = dsum / (float(_H * _W) - ne)
    o_ref[...] = jnp.where(m, -1.0, x - mean)


def _depth_normalize(depth_raw):
    return pl.pallas_call(
        _depth_body,
        out_shape=jax.ShapeDtypeStruct((_B, _H * _W), jnp.float32),
    )(depth_raw)


def kernel(idx, zbuf, dists, features):
    idx_flat = idx.astype(jnp.int32).reshape(_N * _K)
    d_flat = dists.reshape(_N * _K)
    depth_raw = zbuf[..., 0].reshape(_B, _H * _W)
    depth_n = _depth_normalize(depth_raw).reshape(_B, _H, _W, 1)
    images = _sc_render(features, idx_flat, d_flat).reshape(_B, _H, _W, _C)
    return jnp.concatenate([images, depth_n], axis=-1)
